# baseline (speedup 1.0000x reference)
import functools
import jax
import jax.numpy as jnp
from jax import lax
from jax.experimental import pallas as pl
from jax.experimental.pallas import tpu as pltpu
from jax.experimental.pallas import tpu_sc as plsc

L = 16
NB1 = 4096
NB2 = 4096
NB3 = 256
NEG_INF = -1e9


def _splat(x):
    return jnp.broadcast_to(x, (L,))


def _extract(vec, lane_splat):
    io = lax.iota(jnp.int32, L)
    sel = jnp.where(io == lane_splat, vec, jnp.zeros_like(vec))
    return jnp.sum(sel)


def _keys_of(v):
    b = plsc.bitcast(v, jnp.int32)
    return jnp.where(b >= 0, b, b ^ jnp.int32(0x7FFFFFFF))


def _scan_hist(cnt_ref, sum_ref, nb, tg, ts, k_s, pz_s):
    def body(i, carry):
        found, b, g0, s0, m_at, s_at, crun, srun = carry
        c = cnt_ref[pl.ds(i * L, L)]
        s = sum_ref[pl.ds(i * L, L)]
        cincl = crun + plsc.cumsum(c)
        sincl = srun + plsc.cumsum(s)
        gab = _splat(tg) - cincl
        sab = _splat(ts) - sincl
        mask = (gab < _splat(k_s)) & (sab <= _splat(pz_s))
        npop = jnp.sum(plsc.all_reduce_population_count(mask))
        lane = _splat(jnp.sum(plsc.all_reduce_ffs(mask))) >> 4
        hit = (npop > 0) & (found == 0)
        b_new = jnp.where(hit, i * L + (jnp.sum(lane) >> 4), b)
        g0_new = jnp.where(hit, _extract(gab, lane), g0)
        s0_new = jnp.where(hit, _extract(sab, lane), s0)
        m_new = jnp.where(hit, _extract(c, lane), m_at)
        sat_new = jnp.where(hit, _extract(s, lane), s_at)
        found = jnp.where(hit, jnp.int32(1), found)
        crun_new = _extract(cincl, _splat(jnp.int32(L - 1)))
        srun_new = _extract(sincl, _splat(jnp.int32(L - 1)))
        return (found, b_new, g0_new, s0_new, m_new, sat_new,
                crun_new, srun_new)

    init = (jnp.int32(0), jnp.int32(0), jnp.int32(0), jnp.float32(0),
            jnp.int32(0), jnp.float32(0), jnp.int32(0), jnp.float32(0))
    out = lax.fori_loop(0, nb // L, body, init)
    return out[1], out[2], out[3], out[4], out[5]


def _clear_hists(cnt_ref, sum_ref, nb):
    def body(i, carry):
        cnt_ref[pl.ds(i * L, L)] = jnp.zeros((L,), jnp.int32)
        sum_ref[pl.ds(i * L, L)] = jnp.zeros((L,), jnp.float32)
        return carry
    lax.fori_loop(0, nb // L, body, jnp.int32(0))


def make_sc_kernel(B, V):
    info = plsc.get_sparse_core_info()
    NC, NS = info.num_cores, info.num_subcores
    NW = NC * NS
    assert B % NW == 0 and V % L == 0
    RW = B // NW
    NCH = V // L
    mesh = plsc.VectorSubcoreMesh(core_axis_name="c", subcore_axis_name="s")

    @functools.partial(
        pl.kernel,
        mesh=mesh,
        out_type=jax.ShapeDtypeStruct((B, V), jnp.float32),
        scratch_types=[
            pltpu.VMEM((V,), jnp.float32),
            pltpu.VMEM((B,), jnp.float32),
            pltpu.VMEM((B,), jnp.int32),
            pltpu.VMEM((NB1,), jnp.int32),
            pltpu.VMEM((NB1,), jnp.float32),
        ],
        compiler_params=pltpu.CompilerParams(needs_layout_passes=False),
    )
    def sc_kernel(logits_hbm, p_hbm, k_hbm, out_hbm,
                  row_v, p_v, k_v, hcnt, hsum):
        wid = lax.axis_index("s") * NC + lax.axis_index("c")
        pltpu.sync_copy(p_hbm, p_v)
        pltpu.sync_copy(k_hbm, k_v)
        ones16 = jnp.ones((L,), jnp.int32)

        def row_body(j, carry):
            row = wid * RW + j
            pltpu.sync_copy(logits_hbm.at[row], row_v)

            lane = _splat(row & (L - 1))
            p_s = _extract(p_v[pl.ds((row >> 4) << 4, L)], lane)
            k_s = _extract(k_v[pl.ds((row >> 4) << 4, L)], lane)

            def max_body(i, acc):
                return jnp.maximum(acc, row_v[pl.ds(i * L, L)])
            maxvec = lax.fori_loop(0, NCH, max_body,
                                   jnp.full((L,), -jnp.inf, jnp.float32))
            m_s = jnp.max(maxvec)
            m_spl = _splat(m_s)

            _clear_hists(hcnt, hsum, NB1)

            def h1_body(i, zacc):
                v = row_v[pl.ds(i * L, L)]
                key = _keys_of(v)
                e = jnp.exp(v - m_spl)
                bucket = (key >> 20) + 2048
                plsc.addupdate_scatter(hcnt, [bucket], ones16)
                plsc.addupdate_scatter(hsum, [bucket], e)
                return zacc + e
            zacc = lax.fori_loop(0, NCH, h1_body, jnp.zeros((L,), jnp.float32))
            z_s = jnp.sum(zacc)
            pz_s = p_s * z_s

            b1, g1, s1, m1, sa1 = _scan_hist(
                hcnt, hsum, NB1, jnp.int32(V), z_s, k_s, pz_s)

            _clear_hists(hcnt, hsum, NB1)
            b1s = _splat(b1)

            def h2_body(i, carry):
                v = row_v[pl.ds(i * L, L)]
                key = _keys_of(v)
                e = jnp.exp(v - m_spl)
                msk = ((key >> 20) + 2048) == b1s
                bucket = (key >> 8) & 0xFFF
                plsc.addupdate_scatter(hcnt, [bucket], ones16, mask=msk)
                plsc.addupdate_scatter(hsum, [bucket], e, mask=msk)
                return carry
            lax.fori_loop(0, NCH, h2_body, jnp.int32(0))

            b2, g2, s2, m2c, sa2 = _scan_hist(
                hcnt, hsum, NB2, g1 + m1, s1 + sa1, k_s, pz_s)

            _clear_hists(hcnt, hsum, NB1)
            pref = ((b1 - 2048) << 12) | b2
            prefs = _splat(pref)

            def h3_body(i, carry):
                v = row_v[pl.ds(i * L, L)]
                key = _keys_of(v)
                e = jnp.exp(v - m_spl)
                msk = (key >> 8) == prefs
                bucket = key & 0xFF
                plsc.addupdate_scatter(hcnt, [bucket], ones16, mask=msk)
                plsc.addupdate_scatter(hsum, [bucket], e, mask=msk)
                return carry
            lax.fori_loop(0, NCH, h3_body, jnp.int32(0))

            b3, gf, sf, mf, _saf = _scan_hist(
                hcnt, hsum, NB3, g2 + m2c, s2 + sa2, k_s, pz_s)

            kstar = (pref << 8) | b3
            kst_spl = _splat(kstar)
            bst = jnp.where(kst_spl >= 0, kst_spl,
                            kst_spl ^ jnp.int32(0x7FFFFFFF))
            vstar = plsc.bitcast(bst, jnp.float32)
            estar_v = jnp.exp(vstar - m_spl)
            lane0 = _splat(jnp.int32(0))
            estar = _extract(estar_v, lane0)

            mf_f = mf.astype(jnp.float32)
            ratio_v = _splat(pz_s - sf) / estar_v
            ratio_v = jnp.minimum(ratio_v, _splat(mf_f))
            cnt_i = ratio_v.astype(jnp.int32) + 1
            np_in = _extract(jnp.where(estar_v > 0, cnt_i, _splat(mf)), lane0)
            np_in = jnp.maximum(jnp.minimum(np_in, mf), 1)
            n = jnp.minimum(k_s, gf + np_in)
            r = n - gf

            has_masked = n < V
            m2_s = jnp.maximum(m_s, jnp.where(has_masked,
                                              jnp.float32(NEG_INF),
                                              jnp.float32(-jnp.inf)))
            m2_spl = _splat(m2_s)
            u_s = jnp.where(has_masked,
                            _extract(jnp.exp(_splat(jnp.float32(NEG_INF))
                                             - m2_spl), lane0),
                            jnp.float32(0.0))
            scale = _extract(jnp.exp(m_spl - m2_spl), lane0)
            zk = (sf + r.astype(jnp.float32) * estar) * scale \
                + (V - n).astype(jnp.float32) * u_s
            u_spl = _splat(u_s)
            zk_spl = _splat(zk)

            rs = _splat(r)

            def out_body(i, tie_run):
                v = row_v[pl.ds(i * L, L)]
                key = _keys_of(v)
                gt = key > kst_spl
                eqm = key == kst_spl
                incl = plsc.cumsum(jnp.where(eqm, 1, 0).astype(jnp.int32))
                kept = gt | (eqm & ((tie_run + incl) <= rs))
                e2 = jnp.exp(v - m2_spl)
                out = jnp.where(kept, e2, u_spl) / zk_spl
                row_v[pl.ds(i * L, L)] = out
                return tie_run + plsc.all_reduce_population_count(eqm)
            lax.fori_loop(0, NCH, out_body, jnp.zeros((L,), jnp.int32))

            pltpu.sync_copy(row_v, out_hbm.at[row])
            return carry

        lax.fori_loop(0, RW, row_body, jnp.int32(0))

    return sc_kernel


def kernel(logits, top_ps, top_ks):
    B, V = logits.shape
    p2 = top_ps.astype(jnp.float32)
    k2 = top_ks.astype(jnp.int32)
    return make_sc_kernel(B, V)(logits, p2, k2)


# speedup vs baseline: 1.4171x; 1.4171x over previous
import functools
import jax
import jax.numpy as jnp
from jax import lax
from jax.experimental import pallas as pl
from jax.experimental.pallas import tpu as pltpu
from jax.experimental.pallas import tpu_sc as plsc

L = 16
NB1 = 4096
NB2 = 4096
NB3 = 256
NEG_INF = -1e9


def _splat(x):
    return jnp.broadcast_to(x, (L,))


def _extract(vec, lane_splat):
    io = lax.iota(jnp.int32, L)
    sel = jnp.where(io == lane_splat, vec, jnp.zeros_like(vec))
    return jnp.sum(sel)


def _keys_of(v):
    b = plsc.bitcast(v, jnp.int32)
    return jnp.where(b >= 0, b, b ^ jnp.int32(0x7FFFFFFF))


def _scan_hist(cnt_ref, sum_ref, nb, tg, ts, k_s, pz_s):
    kspl = _splat(k_s)
    pzspl = _splat(pz_s)
    tgspl = _splat(tg)
    tsspl = _splat(ts)

    def body(i, carry):
        found, b, cpre, spre, crun, srun = carry
        c = cnt_ref[pl.ds(i * L, L)]
        s = sum_ref[pl.ds(i * L, L)]
        cincl = _splat(crun) + plsc.cumsum(c)
        sincl = _splat(srun) + plsc.cumsum(s)
        mask = ((tgspl - cincl) < kspl) & ((tsspl - sincl) <= pzspl)
        npop = jnp.sum(plsc.all_reduce_population_count(mask))
        l = jnp.sum(plsc.all_reduce_ffs(mask)) >> 4
        hit = (npop > 0) & (found == 0)
        b = jnp.where(hit, i * L + l, b)
        cpre = jnp.where(hit, crun, cpre)
        spre = jnp.where(hit, srun, spre)
        found = jnp.where(hit, jnp.int32(1), found)
        crun = crun + jnp.sum(c)
        srun = srun + jnp.sum(s)
        return (found, b, cpre, spre, crun, srun)

    init = (jnp.int32(0), jnp.int32(0), jnp.int32(0), jnp.float32(0),
            jnp.int32(0), jnp.float32(0))
    _, b, cpre, spre, _, _ = lax.fori_loop(0, nb // L, body, init,
                                           unroll=2)
    base = (b >> 4) << 4
    lane = _splat(b & (L - 1))
    c = cnt_ref[pl.ds(base, L)]
    s = sum_ref[pl.ds(base, L)]
    cincl = _splat(cpre) + plsc.cumsum(c)
    sincl = _splat(spre) + plsc.cumsum(s)
    g0 = tg - _extract(cincl, lane)
    s0 = ts - _extract(sincl, lane)
    m_at = _extract(c, lane)
    s_at = _extract(s, lane)
    return b, g0, s0, m_at, s_at


def _clear_hists(cnt_ref, sum_ref, nb):
    def body(i, carry):
        cnt_ref[pl.ds(i * L, L)] = jnp.zeros((L,), jnp.int32)
        sum_ref[pl.ds(i * L, L)] = jnp.zeros((L,), jnp.float32)
        return carry
    lax.fori_loop(0, nb // L, body, jnp.int32(0), unroll=8)


def make_sc_kernel(B, V):
    info = plsc.get_sparse_core_info()
    NC, NS = info.num_cores, info.num_subcores
    NW = NC * NS
    assert B % NW == 0 and V % L == 0
    RW = B // NW
    NCH = V // L
    mesh = plsc.VectorSubcoreMesh(core_axis_name="c", subcore_axis_name="s")

    @functools.partial(
        pl.kernel,
        mesh=mesh,
        out_type=jax.ShapeDtypeStruct((B, V), jnp.float32),
        scratch_types=[
            pltpu.VMEM((V,), jnp.float32),
            pltpu.VMEM((B,), jnp.float32),
            pltpu.VMEM((B,), jnp.int32),
            pltpu.VMEM((NB1,), jnp.int32),
            pltpu.VMEM((NB1,), jnp.float32),
        ],
        compiler_params=pltpu.CompilerParams(needs_layout_passes=False),
    )
    def sc_kernel(logits_hbm, p_hbm, k_hbm, out_hbm,
                  row_v, p_v, k_v, hcnt, hsum):
        wid = lax.axis_index("s") * NC + lax.axis_index("c")
        pltpu.sync_copy(p_hbm, p_v)
        pltpu.sync_copy(k_hbm, k_v)
        ones16 = jnp.ones((L,), jnp.int32)

        def row_body(j, carry):
            row = wid * RW + j
            pltpu.sync_copy(logits_hbm.at[row], row_v)

            lane = _splat(row & (L - 1))
            p_s = _extract(p_v[pl.ds((row >> 4) << 4, L)], lane)
            k_s = _extract(k_v[pl.ds((row >> 4) << 4, L)], lane)

            def max_body(i, acc):
                return jnp.maximum(acc, row_v[pl.ds(i * L, L)])
            maxvec = lax.fori_loop(0, NCH, max_body,
                                   jnp.full((L,), -jnp.inf, jnp.float32),
                                   unroll=8)
            m_s = jnp.max(maxvec)
            m_spl = _splat(m_s)

            _clear_hists(hcnt, hsum, NB1)

            def h1_body(i, zacc):
                v = row_v[pl.ds(i * L, L)]
                key = _keys_of(v)
                e = jnp.exp(v - m_spl)
                bucket = (key >> 20) + 2048
                plsc.addupdate_scatter(hcnt, [bucket], ones16)
                plsc.addupdate_scatter(hsum, [bucket], e)
                return zacc + e
            zacc = lax.fori_loop(0, NCH, h1_body, jnp.zeros((L,), jnp.float32),
                                 unroll=4)
            z_s = jnp.sum(zacc)
            pz_s = p_s * z_s

            b1, g1, s1, m1, sa1 = _scan_hist(
                hcnt, hsum, NB1, jnp.int32(V), z_s, k_s, pz_s)

            _clear_hists(hcnt, hsum, NB1)
            b1s = _splat(b1)

            def h2_body(i, carry):
                v = row_v[pl.ds(i * L, L)]
                key = _keys_of(v)
                e = jnp.exp(v - m_spl)
                msk = ((key >> 20) + 2048) == b1s
                bucket = (key >> 8) & 0xFFF
                plsc.addupdate_scatter(hcnt, [bucket], ones16, mask=msk)
                plsc.addupdate_scatter(hsum, [bucket], e, mask=msk)
                return carry
            lax.fori_loop(0, NCH, h2_body, jnp.int32(0), unroll=4)

            b2, g2, s2, m2c, sa2 = _scan_hist(
                hcnt, hsum, NB2, g1 + m1, s1 + sa1, k_s, pz_s)

            _clear_hists(hcnt, hsum, NB1)
            pref = ((b1 - 2048) << 12) | b2
            prefs = _splat(pref)

            def h3_body(i, carry):
                v = row_v[pl.ds(i * L, L)]
                key = _keys_of(v)
                e = jnp.exp(v - m_spl)
                msk = (key >> 8) == prefs
                bucket = key & 0xFF
                plsc.addupdate_scatter(hcnt, [bucket], ones16, mask=msk)
                plsc.addupdate_scatter(hsum, [bucket], e, mask=msk)
                return carry
            lax.fori_loop(0, NCH, h3_body, jnp.int32(0), unroll=4)

            b3, gf, sf, mf, _saf = _scan_hist(
                hcnt, hsum, NB3, g2 + m2c, s2 + sa2, k_s, pz_s)

            kstar = (pref << 8) | b3
            kst_spl = _splat(kstar)
            bst = jnp.where(kst_spl >= 0, kst_spl,
                            kst_spl ^ jnp.int32(0x7FFFFFFF))
            vstar = plsc.bitcast(bst, jnp.float32)
            estar_v = jnp.exp(vstar - m_spl)
            lane0 = _splat(jnp.int32(0))
            estar = _extract(estar_v, lane0)

            mf_f = mf.astype(jnp.float32)
            ratio_v = _splat(pz_s - sf) / estar_v
            ratio_v = jnp.minimum(ratio_v, _splat(mf_f))
            cnt_i = ratio_v.astype(jnp.int32) + 1
            np_in = _extract(jnp.where(estar_v > 0, cnt_i, _splat(mf)), lane0)
            np_in = jnp.maximum(jnp.minimum(np_in, mf), 1)
            n = jnp.minimum(k_s, gf + np_in)
            r = n - gf

            has_masked = n < V
            m2_s = jnp.maximum(m_s, jnp.where(has_masked,
                                              jnp.float32(NEG_INF),
                                              jnp.float32(-jnp.inf)))
            m2_spl = _splat(m2_s)
            u_s = jnp.where(has_masked,
                            _extract(jnp.exp(_splat(jnp.float32(NEG_INF))
                                             - m2_spl), lane0),
                            jnp.float32(0.0))
            scale = _extract(jnp.exp(m_spl - m2_spl), lane0)
            zk = (sf + r.astype(jnp.float32) * estar) * scale \
                + (V - n).astype(jnp.float32) * u_s
            inv_zk = jnp.ones((L,), jnp.float32) / _splat(zk)
            u_over_zk = _splat(u_s) * inv_zk

            rs = _splat(r)

            def out_body(i, tie_run):
                v = row_v[pl.ds(i * L, L)]
                key = _keys_of(v)
                gt = key > kst_spl
                eqm = key == kst_spl
                incl = plsc.cumsum(jnp.where(eqm, 1, 0).astype(jnp.int32))
                kept = gt | (eqm & ((tie_run + incl) <= rs))
                e2 = jnp.exp(v - m2_spl)
                out = jnp.where(kept, e2 * inv_zk, u_over_zk)
                row_v[pl.ds(i * L, L)] = out
                return tie_run + plsc.all_reduce_population_count(eqm)
            lax.fori_loop(0, NCH, out_body, jnp.zeros((L,), jnp.int32),
                          unroll=4)

            pltpu.sync_copy(row_v, out_hbm.at[row])
            return carry

        lax.fori_loop(0, RW, row_body, jnp.int32(0))

    return sc_kernel


def kernel(logits, top_ps, top_ks):
    B, V = logits.shape
    p2 = top_ps.astype(jnp.float32)
    k2 = top_ks.astype(jnp.int32)
    return make_sc_kernel(B, V)(logits, p2, k2)


# speedup vs baseline: 1.8086x; 1.2763x over previous
import functools
import jax
import jax.numpy as jnp
from jax import lax
from jax.experimental import pallas as pl
from jax.experimental.pallas import tpu as pltpu
from jax.experimental.pallas import tpu_sc as plsc

L = 16
NB1 = 4096
NB2 = 4096
NB3 = 256
CAP = 2048
NEG_INF = -1e9


def _splat(x):
    return jnp.broadcast_to(x, (L,))


def _extract(vec, lane_splat):
    io = lax.iota(jnp.int32, L)
    sel = jnp.where(io == lane_splat, vec, jnp.zeros_like(vec))
    return jnp.sum(sel)


def _keys_of(v):
    b = plsc.bitcast(v, jnp.int32)
    return jnp.where(b >= 0, b, b ^ jnp.int32(0x7FFFFFFF))


def _scan_hist(cnt_ref, sum_ref, nb, tg, ts, k_s, pz_s):
    kspl = _splat(k_s)
    pzspl = _splat(pz_s)
    tgspl = _splat(tg)
    tsspl = _splat(ts)

    def body(i, carry):
        found, b, cpre, spre, crun, srun = carry
        c = cnt_ref[pl.ds(i * L, L)]
        s = sum_ref[pl.ds(i * L, L)]
        cincl = _splat(crun) + plsc.cumsum(c)
        sincl = _splat(srun) + plsc.cumsum(s)
        mask = ((tgspl - cincl) < kspl) & ((tsspl - sincl) <= pzspl)
        npop = jnp.sum(plsc.all_reduce_population_count(mask))
        l = jnp.sum(plsc.all_reduce_ffs(mask)) >> 4
        hit = (npop > 0) & (found == 0)
        b = jnp.where(hit, i * L + l, b)
        cpre = jnp.where(hit, crun, cpre)
        spre = jnp.where(hit, srun, spre)
        found = jnp.where(hit, jnp.int32(1), found)
        crun = crun + jnp.sum(c)
        srun = srun + jnp.sum(s)
        return (found, b, cpre, spre, crun, srun)

    init = (jnp.int32(0), jnp.int32(0), jnp.int32(0), jnp.float32(0),
            jnp.int32(0), jnp.float32(0))
    _, b, cpre, spre, _, _ = lax.fori_loop(0, nb // L, body, init,
                                           unroll=2)
    base = (b >> 4) << 4
    lane = _splat(b & (L - 1))
    c = cnt_ref[pl.ds(base, L)]
    s = sum_ref[pl.ds(base, L)]
    cincl = _splat(cpre) + plsc.cumsum(c)
    sincl = _splat(spre) + plsc.cumsum(s)
    g0 = tg - _extract(cincl, lane)
    s0 = ts - _extract(sincl, lane)
    m_at = _extract(c, lane)
    s_at = _extract(s, lane)
    return b, g0, s0, m_at, s_at


def _clear_hists(cnt_ref, sum_ref, nb):
    def body(i, carry):
        cnt_ref[pl.ds(i * L, L)] = jnp.zeros((L,), jnp.int32)
        sum_ref[pl.ds(i * L, L)] = jnp.zeros((L,), jnp.float32)
        return carry
    lax.fori_loop(0, nb // L, body, jnp.int32(0), unroll=8)


def make_sc_kernel(B, V):
    info = plsc.get_sparse_core_info()
    NC, NS = info.num_cores, info.num_subcores
    NW = NC * NS
    assert B % NW == 0 and V % L == 0
    RW = B // NW
    NCH = V // L
    mesh = plsc.VectorSubcoreMesh(core_axis_name="c", subcore_axis_name="s")

    @functools.partial(
        pl.kernel,
        mesh=mesh,
        out_type=jax.ShapeDtypeStruct((B, V), jnp.float32),
        scratch_types=[
            pltpu.VMEM((V,), jnp.float32),
            pltpu.VMEM((B,), jnp.float32),
            pltpu.VMEM((B,), jnp.int32),
            pltpu.VMEM((NB1,), jnp.int32),
            pltpu.VMEM((NB1,), jnp.float32),
            pltpu.VMEM((CAP + L,), jnp.float32),
        ],
        compiler_params=pltpu.CompilerParams(needs_layout_passes=False),
    )
    def sc_kernel(logits_hbm, p_hbm, k_hbm, out_hbm,
                  row_v, p_v, k_v, hcnt, hsum, cand_v):
        wid = lax.axis_index("s") * NC + lax.axis_index("c")
        pltpu.sync_copy(p_hbm, p_v)
        pltpu.sync_copy(k_hbm, k_v)
        ones16 = jnp.ones((L,), jnp.int32)

        def row_body(j, carry):
            row = wid * RW + j
            pltpu.sync_copy(logits_hbm.at[row], row_v)

            lane = _splat(row & (L - 1))
            p_s = _extract(p_v[pl.ds((row >> 4) << 4, L)], lane)
            k_s = _extract(k_v[pl.ds((row >> 4) << 4, L)], lane)

            def max_body(i, acc):
                return jnp.maximum(acc, row_v[pl.ds(i * L, L)])
            maxvec = lax.fori_loop(0, NCH, max_body,
                                   jnp.full((L,), -jnp.inf, jnp.float32),
                                   unroll=8)
            m_s = jnp.max(maxvec)
            m_spl = _splat(m_s)

            _clear_hists(hcnt, hsum, NB1)

            def h1_body(i, zacc):
                v = row_v[pl.ds(i * L, L)]
                key = _keys_of(v)
                e = jnp.exp(v - m_spl)
                bucket = (key >> 20) + 2048
                plsc.addupdate_scatter(hcnt, [bucket], ones16)
                plsc.addupdate_scatter(hsum, [bucket], e)
                return zacc + e
            zacc = lax.fori_loop(0, NCH, h1_body, jnp.zeros((L,), jnp.float32),
                                 unroll=4)
            z_s = jnp.sum(zacc)
            pz_s = p_s * z_s

            b1, g1, s1, m1, sa1 = _scan_hist(
                hcnt, hsum, NB1, jnp.int32(V), z_s, k_s, pz_s)

            _clear_hists(hcnt, hsum, NB1)
            b1s = _splat(b1)
            tg2 = g1 + m1
            ts2 = s1 + sa1
            iolane = lax.iota(jnp.int32, L)

            def hist12_of(src_ref, n_ch, m_valid):
                mval = _splat(m_valid)

                def body(i, carry):
                    v = src_ref[pl.ds(i * L, L)]
                    key = _keys_of(v)
                    e = jnp.exp(v - m_spl)
                    msk = (((key >> 20) + 2048) == b1s) \
                        & ((iolane + i * L) < mval)
                    bucket = (key >> 8) & 0xFFF
                    plsc.addupdate_scatter(hcnt, [bucket], ones16, mask=msk)
                    plsc.addupdate_scatter(hsum, [bucket], e, mask=msk)
                    return carry
                return body

            def hist8_of(src_ref, prefs, m_valid):
                mval = _splat(m_valid)

                def body(i, carry):
                    v = src_ref[pl.ds(i * L, L)]
                    key = _keys_of(v)
                    e = jnp.exp(v - m_spl)
                    msk = ((key >> 8) == prefs) & ((iolane + i * L) < mval)
                    bucket = key & 0xFF
                    plsc.addupdate_scatter(hcnt, [bucket], ones16, mask=msk)
                    plsc.addupdate_scatter(hsum, [bucket], e, mask=msk)
                    return carry
                return body

            def refine_small():
                def comp_body(i, off):
                    v = row_v[pl.ds(i * L, L)]
                    key = _keys_of(v)
                    msk = ((key >> 20) + 2048) == b1s
                    plsc.store_compressed(cand_v.at[pl.ds(off, L)], v,
                                          mask=msk)
                    npop = jnp.sum(plsc.all_reduce_population_count(msk))
                    return off + (npop >> 4)
                lax.fori_loop(0, NCH, comp_body, jnp.int32(0), unroll=2)
                nch2 = (m1 + (L - 1)) >> 4
                lax.fori_loop(0, nch2, hist12_of(cand_v, nch2, m1),
                              jnp.int32(0))
                b2, g2, s2, m2c, sa2 = _scan_hist(
                    hcnt, hsum, NB2, tg2, ts2, k_s, pz_s)
                _clear_hists(hcnt, hsum, NB1)
                pref = ((b1 - 2048) << 12) | b2
                lax.fori_loop(0, nch2, hist8_of(cand_v, _splat(pref), m1),
                              jnp.int32(0))
                b3, gf, sf, mf, _saf = _scan_hist(
                    hcnt, hsum, NB3, g2 + m2c, s2 + sa2, k_s, pz_s)
                return pref, b3, gf, sf, mf

            def refine_full():
                lax.fori_loop(0, NCH, hist12_of(row_v, NCH, jnp.int32(V)),
                              jnp.int32(0), unroll=4)
                b2, g2, s2, m2c, sa2 = _scan_hist(
                    hcnt, hsum, NB2, tg2, ts2, k_s, pz_s)
                _clear_hists(hcnt, hsum, NB1)
                pref = ((b1 - 2048) << 12) | b2
                lax.fori_loop(0, NCH, hist8_of(row_v, _splat(pref),
                                               jnp.int32(V)),
                              jnp.int32(0), unroll=4)
                b3, gf, sf, mf, _saf = _scan_hist(
                    hcnt, hsum, NB3, g2 + m2c, s2 + sa2, k_s, pz_s)
                return pref, b3, gf, sf, mf

            pref, b3, gf, sf, mf = lax.cond(
                m1 <= CAP, refine_small, refine_full)

            kstar = (pref << 8) | b3
            kst_spl = _splat(kstar)
            bst = jnp.where(kst_spl >= 0, kst_spl,
                            kst_spl ^ jnp.int32(0x7FFFFFFF))
            vstar = plsc.bitcast(bst, jnp.float32)
            estar_v = jnp.exp(vstar - m_spl)
            lane0 = _splat(jnp.int32(0))
            estar = _extract(estar_v, lane0)

            mf_f = mf.astype(jnp.float32)
            ratio_v = _splat(pz_s - sf) / estar_v
            ratio_v = jnp.minimum(ratio_v, _splat(mf_f))
            cnt_i = ratio_v.astype(jnp.int32) + 1
            np_in = _extract(jnp.where(estar_v > 0, cnt_i, _splat(mf)), lane0)
            np_in = jnp.maximum(jnp.minimum(np_in, mf), 1)
            n = jnp.minimum(k_s, gf + np_in)
            r = n - gf

            has_masked = n < V
            m2_s = jnp.maximum(m_s, jnp.where(has_masked,
                                              jnp.float32(NEG_INF),
                                              jnp.float32(-jnp.inf)))
            m2_spl = _splat(m2_s)
            u_s = jnp.where(has_masked,
                            _extract(jnp.exp(_splat(jnp.float32(NEG_INF))
                                             - m2_spl), lane0),
                            jnp.float32(0.0))
            scale = _extract(jnp.exp(m_spl - m2_spl), lane0)
            zk = (sf + r.astype(jnp.float32) * estar) * scale \
                + (V - n).astype(jnp.float32) * u_s
            inv_zk = jnp.ones((L,), jnp.float32) / _splat(zk)
            u_over_zk = _splat(u_s) * inv_zk

            rs = _splat(r)

            def out_body(i, tie_run):
                v = row_v[pl.ds(i * L, L)]
                key = _keys_of(v)
                gt = key > kst_spl
                eqm = key == kst_spl
                incl = plsc.cumsum(jnp.where(eqm, 1, 0).astype(jnp.int32))
                kept = gt | (eqm & ((tie_run + incl) <= rs))
                e2 = jnp.exp(v - m2_spl)
                out = jnp.where(kept, e2 * inv_zk, u_over_zk)
                row_v[pl.ds(i * L, L)] = out
                return tie_run + plsc.all_reduce_population_count(eqm)
            lax.fori_loop(0, NCH, out_body, jnp.zeros((L,), jnp.int32),
                          unroll=4)

            pltpu.sync_copy(row_v, out_hbm.at[row])
            return carry

        lax.fori_loop(0, RW, row_body, jnp.int32(0))

    return sc_kernel


def kernel(logits, top_ps, top_ks):
    B, V = logits.shape
    p2 = top_ps.astype(jnp.float32)
    k2 = top_ks.astype(jnp.int32)
    return make_sc_kernel(B, V)(logits, p2, k2)


# speedup vs baseline: 1.8800x; 1.0395x over previous
import functools
import jax
import jax.numpy as jnp
from jax import lax
from jax.experimental import pallas as pl
from jax.experimental.pallas import tpu as pltpu
from jax.experimental.pallas import tpu_sc as plsc

L = 16
NB1 = 4096
NB2 = 4096
NB3 = 256
CAP = 2048
NEG_INF = -1e9


def _splat(x):
    return jnp.broadcast_to(x, (L,))


def _extract(vec, lane_splat):
    io = lax.iota(jnp.int32, L)
    sel = jnp.where(io == lane_splat, vec, jnp.zeros_like(vec))
    return jnp.sum(sel)


def _keys_of(v):
    b = plsc.bitcast(v, jnp.int32)
    return jnp.where(b >= 0, b, b ^ jnp.int32(0x7FFFFFFF))


def _scan_hist(cnt_ref, sum_ref, nb, tg, ts, k_s, pz_s):
    kspl = _splat(k_s)
    pzspl = _splat(pz_s)
    tgspl = _splat(tg)
    tsspl = _splat(ts)

    def body(i, carry):
        found, b, cpre, spre, crun, srun = carry
        c = cnt_ref[pl.ds(i * L, L)]
        s = sum_ref[pl.ds(i * L, L)]
        cincl = _splat(crun) + plsc.cumsum(c)
        sincl = _splat(srun) + plsc.cumsum(s)
        mask = ((tgspl - cincl) < kspl) & ((tsspl - sincl) <= pzspl)
        npop = jnp.sum(plsc.all_reduce_population_count(mask))
        l = jnp.sum(plsc.all_reduce_ffs(mask)) >> 4
        hit = (npop > 0) & (found == 0)
        b = jnp.where(hit, i * L + l, b)
        cpre = jnp.where(hit, crun, cpre)
        spre = jnp.where(hit, srun, spre)
        found = jnp.where(hit, jnp.int32(1), found)
        crun = crun + jnp.sum(c)
        srun = srun + jnp.sum(s)
        return (found, b, cpre, spre, crun, srun)

    init = (jnp.int32(0), jnp.int32(0), jnp.int32(0), jnp.float32(0),
            jnp.int32(0), jnp.float32(0))
    _, b, cpre, spre, _, _ = lax.fori_loop(0, nb // L, body, init,
                                           unroll=2)
    base = (b >> 4) << 4
    lane = _splat(b & (L - 1))
    c = cnt_ref[pl.ds(base, L)]
    s = sum_ref[pl.ds(base, L)]
    cincl = _splat(cpre) + plsc.cumsum(c)
    sincl = _splat(spre) + plsc.cumsum(s)
    g0 = tg - _extract(cincl, lane)
    s0 = ts - _extract(sincl, lane)
    m_at = _extract(c, lane)
    s_at = _extract(s, lane)
    return b, g0, s0, m_at, s_at


def _clear_hists(cnt_ref, sum_ref, nb):
    def body(i, carry):
        cnt_ref[pl.ds(i * L, L)] = jnp.zeros((L,), jnp.int32)
        sum_ref[pl.ds(i * L, L)] = jnp.zeros((L,), jnp.float32)
        return carry
    lax.fori_loop(0, nb // L, body, jnp.int32(0), unroll=8)


def make_sc_kernel(B, V):
    info = plsc.get_sparse_core_info()
    NC, NS = info.num_cores, info.num_subcores
    NW = NC * NS
    assert B % NW == 0 and V % L == 0
    RW = B // NW
    NCH = V // L
    mesh = plsc.VectorSubcoreMesh(core_axis_name="c", subcore_axis_name="s")

    @functools.partial(
        pl.kernel,
        mesh=mesh,
        out_type=jax.ShapeDtypeStruct((B, V), jnp.float32),
        scratch_types=[
            pltpu.VMEM((V,), jnp.float32),
            pltpu.VMEM((B,), jnp.float32),
            pltpu.VMEM((B,), jnp.int32),
            pltpu.VMEM((NB1,), jnp.int32),
            pltpu.VMEM((NB1,), jnp.float32),
            pltpu.VMEM((CAP + L,), jnp.float32),
        ],
        compiler_params=pltpu.CompilerParams(needs_layout_passes=False),
    )
    def sc_kernel(logits_hbm, p_hbm, k_hbm, out_hbm,
                  row_v, p_v, k_v, hcnt, hsum, cand_v):
        wid = lax.axis_index("s") * NC + lax.axis_index("c")
        pltpu.sync_copy(p_hbm, p_v)
        pltpu.sync_copy(k_hbm, k_v)
        ones16 = jnp.ones((L,), jnp.int32)

        def row_body(j, carry):
            row = wid * RW + j
            pltpu.sync_copy(logits_hbm.at[row], row_v)

            lane = _splat(row & (L - 1))
            p_s = _extract(p_v[pl.ds((row >> 4) << 4, L)], lane)
            k_s = _extract(k_v[pl.ds((row >> 4) << 4, L)], lane)

            def max_body(i, acc):
                return jnp.maximum(acc, row_v[pl.ds(i * L, L)])
            maxvec = lax.fori_loop(0, NCH, max_body,
                                   jnp.full((L,), -jnp.inf, jnp.float32),
                                   unroll=8)
            m_s = jnp.max(maxvec)
            m_spl = _splat(m_s)

            _clear_hists(hcnt, hsum, NB1)

            def h1_body(i, zacc):
                v = row_v[pl.ds(i * L, L)]
                key = _keys_of(v)
                e = jnp.exp(v - m_spl)
                bucket = (key >> 20) + 2048
                plsc.addupdate_scatter(hcnt, [bucket], ones16)
                plsc.addupdate_scatter(hsum, [bucket], e)
                return zacc + e
            zacc = lax.fori_loop(0, NCH, h1_body, jnp.zeros((L,), jnp.float32),
                                 unroll=8)
            z_s = jnp.sum(zacc)
            pz_s = p_s * z_s

            b1, g1, s1, m1, sa1 = _scan_hist(
                hcnt, hsum, NB1, jnp.int32(V), z_s, k_s, pz_s)

            _clear_hists(hcnt, hsum, NB1)
            b1s = _splat(b1)
            tg2 = g1 + m1
            ts2 = s1 + sa1
            iolane = lax.iota(jnp.int32, L)

            def hist12_of(src_ref, n_ch, m_valid):
                mval = _splat(m_valid)

                def body(i, carry):
                    v = src_ref[pl.ds(i * L, L)]
                    key = _keys_of(v)
                    e = jnp.exp(v - m_spl)
                    msk = (((key >> 20) + 2048) == b1s) \
                        & ((iolane + i * L) < mval)
                    bucket = (key >> 8) & 0xFFF
                    plsc.addupdate_scatter(hcnt, [bucket], ones16, mask=msk)
                    plsc.addupdate_scatter(hsum, [bucket], e, mask=msk)
                    return carry
                return body

            def hist8_of(src_ref, prefs, m_valid):
                mval = _splat(m_valid)

                def body(i, carry):
                    v = src_ref[pl.ds(i * L, L)]
                    key = _keys_of(v)
                    e = jnp.exp(v - m_spl)
                    msk = ((key >> 8) == prefs) & ((iolane + i * L) < mval)
                    bucket = key & 0xFF
                    plsc.addupdate_scatter(hcnt, [bucket], ones16, mask=msk)
                    plsc.addupdate_scatter(hsum, [bucket], e, mask=msk)
                    return carry
                return body

            def refine_small():
                def comp_body(i, off):
                    v = row_v[pl.ds(i * L, L)]
                    key = _keys_of(v)
                    msk = ((key >> 20) + 2048) == b1s
                    plsc.store_compressed(cand_v.at[pl.ds(off, L)], v,
                                          mask=msk)
                    npop = jnp.sum(plsc.all_reduce_population_count(msk))
                    return off + (npop >> 4)
                lax.fori_loop(0, NCH, comp_body, jnp.int32(0), unroll=2)
                nch2 = (m1 + (L - 1)) >> 4
                lax.fori_loop(0, nch2, hist12_of(cand_v, nch2, m1),
                              jnp.int32(0))
                b2, g2, s2, m2c, sa2 = _scan_hist(
                    hcnt, hsum, NB2, tg2, ts2, k_s, pz_s)
                _clear_hists(hcnt, hsum, NB1)
                pref = ((b1 - 2048) << 12) | b2
                lax.fori_loop(0, nch2, hist8_of(cand_v, _splat(pref), m1),
                              jnp.int32(0))
                b3, gf, sf, mf, _saf = _scan_hist(
                    hcnt, hsum, NB3, g2 + m2c, s2 + sa2, k_s, pz_s)
                return pref, b3, gf, sf, mf

            def refine_full():
                lax.fori_loop(0, NCH, hist12_of(row_v, NCH, jnp.int32(V)),
                              jnp.int32(0), unroll=4)
                b2, g2, s2, m2c, sa2 = _scan_hist(
                    hcnt, hsum, NB2, tg2, ts2, k_s, pz_s)
                _clear_hists(hcnt, hsum, NB1)
                pref = ((b1 - 2048) << 12) | b2
                lax.fori_loop(0, NCH, hist8_of(row_v, _splat(pref),
                                               jnp.int32(V)),
                              jnp.int32(0), unroll=4)
                b3, gf, sf, mf, _saf = _scan_hist(
                    hcnt, hsum, NB3, g2 + m2c, s2 + sa2, k_s, pz_s)
                return pref, b3, gf, sf, mf

            pref, b3, gf, sf, mf = lax.cond(
                m1 <= CAP, refine_small, refine_full)

            kstar = (pref << 8) | b3
            kst_spl = _splat(kstar)
            bst = jnp.where(kst_spl >= 0, kst_spl,
                            kst_spl ^ jnp.int32(0x7FFFFFFF))
            vstar = plsc.bitcast(bst, jnp.float32)
            estar_v = jnp.exp(vstar - m_spl)
            lane0 = _splat(jnp.int32(0))
            estar = _extract(estar_v, lane0)

            mf_f = mf.astype(jnp.float32)
            ratio_v = _splat(pz_s - sf) / estar_v
            ratio_v = jnp.minimum(ratio_v, _splat(mf_f))
            cnt_i = ratio_v.astype(jnp.int32) + 1
            np_in = _extract(jnp.where(estar_v > 0, cnt_i, _splat(mf)), lane0)
            np_in = jnp.maximum(jnp.minimum(np_in, mf), 1)
            n = jnp.minimum(k_s, gf + np_in)
            r = n - gf

            has_masked = n < V
            m2_s = jnp.maximum(m_s, jnp.where(has_masked,
                                              jnp.float32(NEG_INF),
                                              jnp.float32(-jnp.inf)))
            m2_spl = _splat(m2_s)
            u_s = jnp.where(has_masked,
                            _extract(jnp.exp(_splat(jnp.float32(NEG_INF))
                                             - m2_spl), lane0),
                            jnp.float32(0.0))
            scale = _extract(jnp.exp(m_spl - m2_spl), lane0)
            zk = (sf + r.astype(jnp.float32) * estar) * scale \
                + (V - n).astype(jnp.float32) * u_s
            inv_zk = jnp.ones((L,), jnp.float32) / _splat(zk)
            u_over_zk = _splat(u_s) * inv_zk

            rs = _splat(r)

            def out_simple(i, carry):
                v = row_v[pl.ds(i * L, L)]
                key = _keys_of(v)
                kept = key >= kst_spl
                e2 = jnp.exp(v - m2_spl)
                row_v[pl.ds(i * L, L)] = jnp.where(kept, e2 * inv_zk,
                                                   u_over_zk)
                return carry

            def out_tie(i, tie_run):
                v = row_v[pl.ds(i * L, L)]
                key = _keys_of(v)
                gt = key > kst_spl
                eqm = key == kst_spl
                incl = plsc.cumsum(jnp.where(eqm, 1, 0).astype(jnp.int32))
                kept = gt | (eqm & ((tie_run + incl) <= rs))
                e2 = jnp.exp(v - m2_spl)
                out = jnp.where(kept, e2 * inv_zk, u_over_zk)
                row_v[pl.ds(i * L, L)] = out
                return tie_run + plsc.all_reduce_population_count(eqm)

            def do_simple():
                lax.fori_loop(0, NCH, out_simple, jnp.int32(0), unroll=8)
                return jnp.int32(0)

            def do_tie():
                lax.fori_loop(0, NCH, out_tie, jnp.zeros((L,), jnp.int32),
                              unroll=4)
                return jnp.int32(0)

            lax.cond(r == mf, do_simple, do_tie)

            pltpu.sync_copy(row_v, out_hbm.at[row])
            return carry

        lax.fori_loop(0, RW, row_body, jnp.int32(0))

    return sc_kernel


def kernel(logits, top_ps, top_ks):
    B, V = logits.shape
    p2 = top_ps.astype(jnp.float32)
    k2 = top_ks.astype(jnp.int32)
    return make_sc_kernel(B, V)(logits, p2, k2)


# speedup vs baseline: 1.8822x; 1.0012x over previous
import functools
import jax
import jax.numpy as jnp
from jax import lax
from jax.experimental import pallas as pl
from jax.experimental.pallas import tpu as pltpu
from jax.experimental.pallas import tpu_sc as plsc

L = 16
NB1 = 4096
NB2 = 4096
NB3 = 256
CAP = 2048
NEG_INF = -1e9


def _splat(x):
    return jnp.broadcast_to(x, (L,))


def _extract(vec, lane_splat):
    io = lax.iota(jnp.int32, L)
    sel = jnp.where(io == lane_splat, vec, jnp.zeros_like(vec))
    return jnp.sum(sel)


def _keys_of(v):
    b = plsc.bitcast(v, jnp.int32)
    return jnp.where(b >= 0, b, b ^ jnp.int32(0x7FFFFFFF))


def _scan_hist(cnt_ref, sum_ref, nb, tg, ts, k_s, pz_s):
    kspl = _splat(k_s)
    pzspl = _splat(pz_s)
    tgspl = _splat(tg)
    tsspl = _splat(ts)

    def body(i, carry):
        found, b, cpre, spre, crun, srun = carry
        c = cnt_ref[pl.ds(i * L, L)]
        s = sum_ref[pl.ds(i * L, L)]
        cincl = _splat(crun) + plsc.cumsum(c)
        sincl = _splat(srun) + plsc.cumsum(s)
        mask = ((tgspl - cincl) < kspl) & ((tsspl - sincl) <= pzspl)
        npop = jnp.sum(plsc.all_reduce_population_count(mask))
        l = jnp.sum(plsc.all_reduce_ffs(mask)) >> 4
        hit = (npop > 0) & (found == 0)
        b = jnp.where(hit, i * L + l, b)
        cpre = jnp.where(hit, crun, cpre)
        spre = jnp.where(hit, srun, spre)
        found = jnp.where(hit, jnp.int32(1), found)
        crun = crun + jnp.sum(c)
        srun = srun + jnp.sum(s)
        return (found, b, cpre, spre, crun, srun)

    init = (jnp.int32(0), jnp.int32(0), jnp.int32(0), jnp.float32(0),
            jnp.int32(0), jnp.float32(0))
    _, b, cpre, spre, _, _ = lax.fori_loop(0, nb // L, body, init,
                                           unroll=2)
    base = (b >> 4) << 4
    lane = _splat(b & (L - 1))
    c = cnt_ref[pl.ds(base, L)]
    s = sum_ref[pl.ds(base, L)]
    cincl = _splat(cpre) + plsc.cumsum(c)
    sincl = _splat(spre) + plsc.cumsum(s)
    g0 = tg - _extract(cincl, lane)
    s0 = ts - _extract(sincl, lane)
    m_at = _extract(c, lane)
    s_at = _extract(s, lane)
    return b, g0, s0, m_at, s_at


def _clear_hists(cnt_ref, sum_ref, nb):
    def body(i, carry):
        cnt_ref[pl.ds(i * L, L)] = jnp.zeros((L,), jnp.int32)
        sum_ref[pl.ds(i * L, L)] = jnp.zeros((L,), jnp.float32)
        return carry
    lax.fori_loop(0, nb // L, body, jnp.int32(0), unroll=8)


def make_sc_kernel(B, V):
    info = plsc.get_sparse_core_info()
    NC, NS = info.num_cores, info.num_subcores
    NW = NC * NS
    assert B % NW == 0 and V % L == 0
    RW = B // NW
    NCH = V // L
    mesh = plsc.VectorSubcoreMesh(core_axis_name="c", subcore_axis_name="s")

    @functools.partial(
        pl.kernel,
        mesh=mesh,
        out_type=jax.ShapeDtypeStruct((B, V), jnp.float32),
        scratch_types=[
            pltpu.VMEM((V,), jnp.float32),
            pltpu.VMEM((B,), jnp.float32),
            pltpu.VMEM((B,), jnp.int32),
            pltpu.VMEM((NB1,), jnp.int32),
            pltpu.VMEM((NB1,), jnp.float32),
            pltpu.VMEM((CAP + L,), jnp.float32),
        ],
        compiler_params=pltpu.CompilerParams(needs_layout_passes=False),
    )
    def sc_kernel(logits_hbm, p_hbm, k_hbm, out_hbm,
                  row_v, p_v, k_v, hcnt, hsum, cand_v):
        wid = lax.axis_index("s") * NC + lax.axis_index("c")
        pltpu.sync_copy(p_hbm, p_v)
        pltpu.sync_copy(k_hbm, k_v)
        ones16 = jnp.ones((L,), jnp.int32)

        def row_body(j, carry):
            row = wid * RW + j
            pltpu.sync_copy(logits_hbm.at[row], row_v)

            lane = _splat(row & (L - 1))
            p_s = _extract(p_v[pl.ds((row >> 4) << 4, L)], lane)
            k_s = _extract(k_v[pl.ds((row >> 4) << 4, L)], lane)

            def max_body(i, acc):
                return jnp.maximum(acc, row_v[pl.ds(i * L, L)])
            maxvec = lax.fori_loop(0, NCH, max_body,
                                   jnp.full((L,), -jnp.inf, jnp.float32),
                                   unroll=8)
            m_s = jnp.max(maxvec)
            m_spl = _splat(m_s)

            _clear_hists(hcnt, hsum, NB1)

            def h1_body(i, zacc):
                v = row_v[pl.ds(i * L, L)]
                key = _keys_of(v)
                e = jnp.exp(v - m_spl)
                bucket = (key >> 20) + 2048
                plsc.addupdate_scatter(hcnt, [bucket], ones16)
                plsc.addupdate_scatter(hsum, [bucket], e)
                return zacc + e
            zacc = lax.fori_loop(0, NCH, h1_body, jnp.zeros((L,), jnp.float32),
                                 unroll=8)
            z_s = jnp.sum(zacc)
            pz_s = p_s * z_s

            b1, g1, s1, m1, sa1 = _scan_hist(
                hcnt, hsum, NB1, jnp.int32(V), z_s, k_s, pz_s)

            _clear_hists(hcnt, hsum, NB1)
            b1s = _splat(b1)
            tg2 = g1 + m1
            ts2 = s1 + sa1
            iolane = lax.iota(jnp.int32, L)

            def hist12_of(src_ref, n_ch, m_valid):
                mval = _splat(m_valid)

                def body(i, carry):
                    v = src_ref[pl.ds(i * L, L)]
                    key = _keys_of(v)
                    e = jnp.exp(v - m_spl)
                    msk = (((key >> 20) + 2048) == b1s) \
                        & ((iolane + i * L) < mval)
                    bucket = (key >> 8) & 0xFFF
                    plsc.addupdate_scatter(hcnt, [bucket], ones16, mask=msk)
                    plsc.addupdate_scatter(hsum, [bucket], e, mask=msk)
                    return carry
                return body

            def hist8_of(src_ref, prefs, m_valid):
                mval = _splat(m_valid)

                def body(i, carry):
                    v = src_ref[pl.ds(i * L, L)]
                    key = _keys_of(v)
                    e = jnp.exp(v - m_spl)
                    msk = ((key >> 8) == prefs) & ((iolane + i * L) < mval)
                    bucket = key & 0xFF
                    plsc.addupdate_scatter(hcnt, [bucket], ones16, mask=msk)
                    plsc.addupdate_scatter(hsum, [bucket], e, mask=msk)
                    return carry
                return body

            def refine_small():
                def comp_body(i, off):
                    v = row_v[pl.ds(i * L, L)]
                    key = _keys_of(v)
                    msk = ((key >> 20) + 2048) == b1s
                    plsc.store_compressed(cand_v.at[pl.ds(off, L)], v,
                                          mask=msk)
                    npop = jnp.sum(plsc.all_reduce_population_count(msk))
                    return off + (npop >> 4)
                lax.fori_loop(0, NCH, comp_body, jnp.int32(0), unroll=2)
                nch2 = (m1 + (L - 1)) >> 4
                lax.fori_loop(0, nch2, hist12_of(cand_v, nch2, m1),
                              jnp.int32(0))
                b2, g2, s2, m2c, sa2 = _scan_hist(
                    hcnt, hsum, NB2, tg2, ts2, k_s, pz_s)
                _clear_hists(hcnt, hsum, NB3)
                pref = ((b1 - 2048) << 12) | b2
                lax.fori_loop(0, nch2, hist8_of(cand_v, _splat(pref), m1),
                              jnp.int32(0))
                b3, gf, sf, mf, _saf = _scan_hist(
                    hcnt, hsum, NB3, g2 + m2c, s2 + sa2, k_s, pz_s)
                return pref, b3, gf, sf, mf

            def refine_full():
                lax.fori_loop(0, NCH, hist12_of(row_v, NCH, jnp.int32(V)),
                              jnp.int32(0), unroll=4)
                b2, g2, s2, m2c, sa2 = _scan_hist(
                    hcnt, hsum, NB2, tg2, ts2, k_s, pz_s)
                _clear_hists(hcnt, hsum, NB3)
                pref = ((b1 - 2048) << 12) | b2
                lax.fori_loop(0, NCH, hist8_of(row_v, _splat(pref),
                                               jnp.int32(V)),
                              jnp.int32(0), unroll=4)
                b3, gf, sf, mf, _saf = _scan_hist(
                    hcnt, hsum, NB3, g2 + m2c, s2 + sa2, k_s, pz_s)
                return pref, b3, gf, sf, mf

            pref, b3, gf, sf, mf = lax.cond(
                m1 <= CAP, refine_small, refine_full)

            kstar = (pref << 8) | b3
            kst_spl = _splat(kstar)
            bst = jnp.where(kst_spl >= 0, kst_spl,
                            kst_spl ^ jnp.int32(0x7FFFFFFF))
            vstar = plsc.bitcast(bst, jnp.float32)
            estar_v = jnp.exp(vstar - m_spl)
            lane0 = _splat(jnp.int32(0))
            estar = _extract(estar_v, lane0)

            mf_f = mf.astype(jnp.float32)
            ratio_v = _splat(pz_s - sf) / estar_v
            ratio_v = jnp.minimum(ratio_v, _splat(mf_f))
            cnt_i = ratio_v.astype(jnp.int32) + 1
            np_in = _extract(jnp.where(estar_v > 0, cnt_i, _splat(mf)), lane0)
            np_in = jnp.maximum(jnp.minimum(np_in, mf), 1)
            n = jnp.minimum(k_s, gf + np_in)
            r = n - gf

            has_masked = n < V
            m2_s = jnp.maximum(m_s, jnp.where(has_masked,
                                              jnp.float32(NEG_INF),
                                              jnp.float32(-jnp.inf)))
            m2_spl = _splat(m2_s)
            u_s = jnp.where(has_masked,
                            _extract(jnp.exp(_splat(jnp.float32(NEG_INF))
                                             - m2_spl), lane0),
                            jnp.float32(0.0))
            scale = _extract(jnp.exp(m_spl - m2_spl), lane0)
            zk = (sf + r.astype(jnp.float32) * estar) * scale \
                + (V - n).astype(jnp.float32) * u_s
            inv_zk = jnp.ones((L,), jnp.float32) / _splat(zk)
            u_over_zk = _splat(u_s) * inv_zk

            rs = _splat(r)

            def out_simple(i, carry):
                v = row_v[pl.ds(i * L, L)]
                key = _keys_of(v)
                kept = key >= kst_spl
                e2 = jnp.exp(v - m2_spl)
                row_v[pl.ds(i * L, L)] = jnp.where(kept, e2 * inv_zk,
                                                   u_over_zk)
                return carry

            def out_tie(i, tie_run):
                v = row_v[pl.ds(i * L, L)]
                key = _keys_of(v)
                gt = key > kst_spl
                eqm = key == kst_spl
                incl = plsc.cumsum(jnp.where(eqm, 1, 0).astype(jnp.int32))
                kept = gt | (eqm & ((tie_run + incl) <= rs))
                e2 = jnp.exp(v - m2_spl)
                out = jnp.where(kept, e2 * inv_zk, u_over_zk)
                row_v[pl.ds(i * L, L)] = out
                return tie_run + plsc.all_reduce_population_count(eqm)

            def do_simple():
                lax.fori_loop(0, NCH, out_simple, jnp.int32(0), unroll=8)
                return jnp.int32(0)

            def do_tie():
                lax.fori_loop(0, NCH, out_tie, jnp.zeros((L,), jnp.int32),
                              unroll=4)
                return jnp.int32(0)

            lax.cond(r == mf, do_simple, do_tie)

            pltpu.sync_copy(row_v, out_hbm.at[row])
            return carry

        lax.fori_loop(0, RW, row_body, jnp.int32(0))

    return sc_kernel


def kernel(logits, top_ps, top_ks):
    B, V = logits.shape
    p2 = top_ps.astype(jnp.float32)
    k2 = top_ks.astype(jnp.int32)
    return make_sc_kernel(B, V)(logits, p2, k2)
